# Initial kernel scaffold; baseline (speedup 1.0000x reference)
import jax
import jax.numpy as jnp
from jax.experimental import pallas as pl


def kernel(x, edge_index, edge_attr, We, be, W1, b1, W2, b2, gamma, beta):
    raise NotImplementedError("write your pallas kernel here")


# speedup vs baseline: 1.8770x; 1.8770x over previous
import functools

import jax
import jax.numpy as jnp
from jax import lax
from jax.experimental import pallas as pl
from jax.experimental.pallas import tpu as pltpu
from jax.experimental.pallas import tpu_sc as plsc

_NC = 2
_NS = 16
_L = 16
_NW = _NC * _NS

_C = 48
_G = 8
_NCH = 216
_NPAD = 10240


def _edge_mm_body(ea_ref, we_ref, be_ref, out_ref):
    out_ref[...] = (
        jnp.dot(ea_ref[...], we_ref[...], preferred_element_type=jnp.float32)
        + be_ref[...]
    )


def _edge_project(edge_attr, We, be, EP):
    E, ED = edge_attr.shape
    D = We.shape[1]
    BE = 4096
    grid = EP // BE
    return pl.pallas_call(
        _edge_mm_body,
        grid=(grid,),
        in_specs=[
            pl.BlockSpec((BE, ED), lambda i: (i, 0)),
            pl.BlockSpec((ED, D), lambda i: (0, 0)),
            pl.BlockSpec((1, D), lambda i: (0, 0)),
        ],
        out_specs=pl.BlockSpec((BE, D), lambda i: (i, 0)),
        out_shape=jax.ShapeDtypeStruct((EP, D), jnp.float32),
    )(edge_attr, We, be.reshape(1, D))


def _make_sc_agg(N, D, EP):
    RT = _NPAD // _NS
    mesh = plsc.VectorSubcoreMesh(core_axis_name="c", subcore_axis_name="s",
                                  num_cores=_NC, num_subcores=_NS)

    @functools.partial(
        pl.kernel,
        mesh=mesh,
        out_type=jax.ShapeDtypeStruct((_NC * _NPAD, D), jnp.float32),
        scratch_types=[
            pltpu.VMEM((_G, _C), jnp.int32),
            pltpu.VMEM((_G, _C), jnp.int32),
            pltpu.VMEM((3, _C, D), jnp.float32),
            pltpu.VMEM((3, _C, D), jnp.float32),
            pltpu.VMEM_SHARED((_NPAD, D), jnp.float32),
            pltpu.SemaphoreType.DMA,
            pltpu.SemaphoreType.DMA,
            pltpu.SemaphoreType.DMA,
            pltpu.SemaphoreType.DMA,
            pltpu.SemaphoreType.DMA,
            pltpu.SemaphoreType.DMA,
            pltpu.SemaphoreType.DMA,
            pltpu.SemaphoreType.DMA,
            pltpu.SemaphoreType.DMA,
        ],
    )
    def sc_agg(x_hbm, srcb_hbm, dstb_hbm, e_hbm, out_hbm,
               idxs, idxd, xv, ev, agg,
               se0, se1, se2, sg0, sg1, sg2, ss0, ss1, ss2):
        sems_e = (se0, se1, se2)
        sems_g = (sg0, sg1, sg2)
        sems_s = (ss0, ss1, ss2)
        c = lax.axis_index("c")
        s = lax.axis_index("s")
        wid = c * _NS + s
        ibase = wid * _NCH

        @plsc.parallel_loop(0, _C, unroll=4)
        def zrow(r):
            for cc in range(D // _L):
                xv[0, r, pl.ds(cc * _L, _L)] = jnp.zeros((_L,), jnp.float32)

        nfull = RT // _C
        for k in range(nfull):
            pltpu.sync_copy(
                xv.at[0],
                agg.at[pl.ds(pl.multiple_of(s * RT + k * _C, 8), _C)])
        rem = RT - nfull * _C
        if rem:
            pltpu.sync_copy(
                xv.at[0, pl.ds(0, rem)],
                agg.at[pl.ds(pl.multiple_of(s * RT + nfull * _C, 8), rem)])
        plsc.subcore_barrier()

        def issue_loads(g, k):
            b = k % 3
            erow = pl.multiple_of((ibase + g * _G + k) * _C, 8)
            de = pltpu.async_copy(e_hbm.at[pl.ds(erow, _C)], ev.at[b],
                                  sems_e[b])
            dg = pltpu.async_copy(x_hbm.at[idxs.at[k]], xv.at[b], sems_g[b])
            return de, dg

        def group(g, carry):
            row0 = pl.multiple_of(ibase + g * _G, 8)
            pltpu.sync_copy(srcb_hbm.at[pl.ds(row0, _G)], idxs)
            pltpu.sync_copy(dstb_hbm.at[pl.ds(row0, _G)], idxd)

            loads = {0: issue_loads(g, 0), 1: issue_loads(g, 1)}
            scat = {}
            for k in range(_G):
                b = k % 3
                de, dg = loads.pop(k)
                de.wait()
                dg.wait()

                @plsc.parallel_loop(0, _C, unroll=2)
                def rowfn(r):
                    for cc in range(D // _L):
                        sl = pl.ds(cc * _L, _L)
                        xv[b, r, sl] = jnp.maximum(
                            xv[b, r, sl] + ev[b, r, sl], 0.0)

                scat[k] = pltpu.async_copy(
                    xv.at[b], agg.at[idxd.at[k]], sems_s[b], add=True)
                if k + 2 < _G:
                    if k >= 1:
                        scat.pop(k - 1).wait()
                    loads[k + 2] = issue_loads(g, k + 2)
            for k in sorted(scat):
                scat.pop(k).wait()
            return carry

        lax.fori_loop(0, _NCH // _G, group, 0)
        plsc.subcore_barrier()

        pltpu.sync_copy(
            agg.at[pl.ds(pl.multiple_of(s * RT, 8), RT)],
            out_hbm.at[pl.ds(pl.multiple_of(c * _NPAD + s * RT, 8), RT)])

    return sc_agg


def _mlp_body(x_ref, p0_ref, p1_ref, w1_ref, b1_ref, w2_ref, b2_ref,
              g_ref, bb_ref, o_ref):
    h = x_ref[...] + p0_ref[0] + p1_ref[0]
    t = jnp.maximum(
        jnp.dot(h, w1_ref[...], preferred_element_type=jnp.float32)
        + b1_ref[...], 0.0)
    h2 = (jnp.dot(t, w2_ref[...], preferred_element_type=jnp.float32)
          + b2_ref[...])
    mu = jnp.mean(h2, axis=-1, keepdims=True)
    var = jnp.mean((h2 - mu) ** 2, axis=-1, keepdims=True)
    hn = (h2 - mu) * lax.rsqrt(var + 1e-5) * g_ref[...] + bb_ref[...]
    o_ref[...] = jnp.maximum(hn, 0.0)


def _node_update(x, parts3, W1, b1, W2, b2, gamma, beta):
    N, D = x.shape
    BN = 2000
    grid = N // BN
    full = lambda i: (0, 0)
    return pl.pallas_call(
        _mlp_body,
        grid=(grid,),
        in_specs=[
            pl.BlockSpec((BN, D), lambda i: (i, 0)),
            pl.BlockSpec((1, BN, D), lambda i: (0, i, 0)),
            pl.BlockSpec((1, BN, D), lambda i: (1, i, 0)),
            pl.BlockSpec((D, D), full),
            pl.BlockSpec((1, D), full),
            pl.BlockSpec((D, D), full),
            pl.BlockSpec((1, D), full),
            pl.BlockSpec((1, D), full),
            pl.BlockSpec((1, D), full),
        ],
        out_specs=pl.BlockSpec((BN, D), lambda i: (i, 0)),
        out_shape=jax.ShapeDtypeStruct((N, D), jnp.float32),
    )(x, parts3, parts3, W1, b1.reshape(1, D), W2, b2.reshape(1, D),
      gamma.reshape(1, D), beta.reshape(1, D))


def kernel(x, edge_index, edge_attr, We, be, W1, b1, W2, b2, gamma, beta):
    N, D = x.shape
    E, ED = edge_attr.shape
    EP = _NW * _NCH * _C
    pad = EP - E

    ea_p = jnp.concatenate([edge_attr, jnp.zeros((pad, ED), jnp.float32)])
    e = _edge_project(ea_p, We, be, EP)

    src2d = jnp.concatenate(
        [edge_index[0], jnp.zeros((pad,), jnp.int32)]).reshape(EP // _C, _C)
    dst2d = jnp.concatenate(
        [edge_index[1], jnp.full((pad,), N, jnp.int32)]).reshape(EP // _C, _C)

    parts = _make_sc_agg(N, D, EP)(x, src2d, dst2d, e)
    parts3 = parts.reshape(_NC, _NPAD, D)
    return _node_update(x, parts3, W1, b1, W2, b2, gamma, beta)
